# baseline (speedup 1.0000x reference)
import functools

import jax
import jax.numpy as jnp
from jax import lax
from jax.experimental import pallas as pl
from jax.experimental.pallas import tpu as pltpu
from jax.experimental.pallas import tpu_sc as plsc

L = 16
NC = 2
NS = 16
NW = NC * NS

HIDDEN = 64
EPS = 1e-6

CHUNK_ROWS = 4
CHUNK = CHUNK_ROWS * 128
GROUPS = CHUNK // L


def _rsqrt(x):
    i = lax.bitcast_convert_type(x, jnp.int32)
    i = jnp.int32(0x5F3759DF) - lax.shift_right_arithmetic(i, 1)
    y = lax.bitcast_convert_type(i, jnp.float32)
    for _ in range(3):
        y = y * (1.5 - 0.5 * x * y * y)
    return y


def _make_kernel(n_tokens):
    rows_per_tile = n_tokens // 128 // NW
    chunks = rows_per_tile // CHUNK_ROWS
    mesh = plsc.VectorSubcoreMesh(core_axis_name="c", subcore_axis_name="s")

    @functools.partial(
        pl.kernel,
        mesh=mesh,
        compiler_params=pltpu.CompilerParams(
            needs_layout_passes=False, use_tc_tiling_on_sc=False
        ),
        out_type=jax.ShapeDtypeStruct((n_tokens, HIDDEN), jnp.float32),
        scratch_types=[
            pltpu.VMEM((CHUNK_ROWS, 128), jnp.int32),
            pltpu.VMEM((CHUNK, HIDDEN), jnp.float32),
            pltpu.VMEM((CHUNK,), jnp.float32),
            pltpu.VMEM((CHUNK,), jnp.float32),
            pltpu.VMEM((HIDDEN,), jnp.float32),
            pltpu.VMEM((HIDDEN,), jnp.float32),
            pltpu.SemaphoreType.DMA,
        ],
    )
    def ln_embed(idx_hbm, table_hbm, gamma_hbm, beta_hbm, out_hbm,
                 idx_v, rows_v, mean_v, rstd_v, gamma_v, beta_v, sem):
        wid = lax.axis_index("s") * NC + lax.axis_index("c")
        pltpu.sync_copy(gamma_hbm, gamma_v)
        pltpu.sync_copy(beta_hbm, beta_v)
        g_regs = [gamma_v[pl.ds(j * L, L)] for j in range(HIDDEN // L)]
        b_regs = [beta_v[pl.ds(j * L, L)] for j in range(HIDDEN // L)]
        row0 = wid * rows_per_tile
        iota = lax.iota(jnp.int32, L)

        def chunk_body(ci, _):
            r = row0 + ci * CHUNK_ROWS
            pltpu.sync_copy(idx_hbm.at[pl.ds(r, CHUNK_ROWS)], idx_v)
            cps = [
                pltpu.async_copy(
                    table_hbm.at[idx_v.at[j]],
                    rows_v.at[pl.ds(j * 128, 128)],
                    sem,
                )
                for j in range(CHUNK_ROWS)
            ]
            for cp in cps:
                cp.wait()

            def grp(gi, _):
                t0 = gi * L
                idx_t = t0 + iota
                s = [jnp.zeros((L,), jnp.float32) for _ in range(4)]
                ss = [jnp.zeros((L,), jnp.float32) for _ in range(4)]
                for h in range(HIDDEN):
                    v = plsc.load_gather(
                        rows_v,
                        [idx_t, jnp.full((L,), h, dtype=jnp.int32)],
                    )
                    s[h % 4] = s[h % 4] + v
                    ss[h % 4] = ss[h % 4] + v * v
                stot = (s[0] + s[1]) + (s[2] + s[3])
                sstot = (ss[0] + ss[1]) + (ss[2] + ss[3])
                mean = stot * (1.0 / HIDDEN)
                var = sstot * (1.0 / HIDDEN) - mean * mean
                mean_v[pl.ds(t0, L)] = mean
                rstd_v[pl.ds(t0, L)] = _rsqrt(var + EPS)
                return 0

            lax.fori_loop(0, GROUPS, grp, 0)

            def tok(t, _):
                tsplat = jnp.full((L,), t, dtype=jnp.int32)
                mb = plsc.load_gather(mean_v, [tsplat])
                rb = plsc.load_gather(rstd_v, [tsplat])
                for j in range(HIDDEN // L):
                    v = rows_v[t, pl.ds(j * L, L)]
                    rows_v[t, pl.ds(j * L, L)] = (
                        (v - mb) * rb * g_regs[j] + b_regs[j]
                    )
                return 0

            lax.fori_loop(0, CHUNK, tok, 0)

            pltpu.sync_copy(rows_v, out_hbm.at[pl.ds(r * 128, CHUNK)])
            return 0

        lax.fori_loop(0, chunks, chunk_body, 0)

    return ln_embed


def kernel(input_tokens, table, gamma, beta):
    b, l = input_tokens.shape
    n = b * l
    idx = input_tokens.reshape(n // 128, 128).astype(jnp.int32)
    out = _make_kernel(n)(idx, table, gamma, beta)
    return out.reshape(b, l, HIDDEN)
